# baseline (speedup 1.0000x reference)
import jax
import jax.numpy as jnp
from jax import lax
from jax.experimental import pallas as pl
from jax.experimental.pallas import tpu as pltpu
from jax.experimental.pallas import tpu_sc as plsc

NC = 2
NS = 16
L = 16
NW = NC * NS


def _sc_body(bpw, dim, vi_hbm, vj_hbm, ft_hbm, st_hbm, ct_hbm,
             out1_hbm, out2_hbm,
             idx_i, idx_j, ra, rb, rc, rd, o1, o2, sem):
    wid = lax.axis_index("s") * NC + lax.axis_index("c")
    base = wid * bpw
    pltpu.sync_copy(vi_hbm.at[pl.ds(base, bpw)], idx_i)
    pltpu.sync_copy(vj_hbm.at[pl.ds(base, bpw)], idx_j)

    copies = []
    for d in range(dim):
        copies.append(pltpu.async_copy(ft_hbm.at[d].at[idx_i], ra.at[d], sem))
        copies.append(pltpu.async_copy(ft_hbm.at[d].at[idx_j], rb.at[d], sem))
        copies.append(pltpu.async_copy(st_hbm.at[d].at[idx_i], rc.at[d], sem))
        copies.append(pltpu.async_copy(ct_hbm.at[d].at[idx_j], rd.at[d], sem))
    for c in copies:
        c.wait()

    def group(g, carry):
        gsl = pl.ds(g * L, L)
        acc1 = jnp.zeros((L,), jnp.float32)
        acc2 = jnp.zeros((L,), jnp.float32)
        for d in range(dim):
            acc1 = acc1 + ra[d, gsl] * rb[d, gsl]
            acc2 = acc2 + rc[d, gsl] * rd[d, gsl]
        o1[gsl] = acc1
        o2[gsl] = acc2
        return carry

    lax.fori_loop(0, bpw // L, group, 0)
    pltpu.sync_copy(o1, out1_hbm.at[pl.ds(base, bpw)])
    pltpu.sync_copy(o2, out2_hbm.at[pl.ds(base, bpw)])


def kernel(v_i, v_j, first_table, second_table, context_table):
    batch = v_i.shape[0]
    nodes, dim = first_table.shape
    assert batch % (NW * L) == 0 and dim == L
    bpw = batch // NW
    v_i = v_i.astype(jnp.int32)
    v_j = v_j.astype(jnp.int32)
    ftt = first_table.T
    stt = second_table.T
    ctt = context_table.T

    mesh = plsc.VectorSubcoreMesh(core_axis_name="c", subcore_axis_name="s")
    f = pl.kernel(
        lambda *refs: _sc_body(bpw, dim, *refs),
        out_type=(
            jax.ShapeDtypeStruct((batch,), jnp.float32),
            jax.ShapeDtypeStruct((batch,), jnp.float32),
        ),
        mesh=mesh,
        compiler_params=pltpu.CompilerParams(
            needs_layout_passes=False, use_tc_tiling_on_sc=False
        ),
        scratch_types=[
            pltpu.VMEM((bpw,), jnp.int32),
            pltpu.VMEM((bpw,), jnp.int32),
            pltpu.VMEM((dim, bpw), jnp.float32),
            pltpu.VMEM((dim, bpw), jnp.float32),
            pltpu.VMEM((dim, bpw), jnp.float32),
            pltpu.VMEM((dim, bpw), jnp.float32),
            pltpu.VMEM((bpw,), jnp.float32),
            pltpu.VMEM((bpw,), jnp.float32),
            pltpu.SemaphoreType.DMA,
        ],
    )
    first, second = f(v_i, v_j, ftt, stt, ctt)
    return (first, second)
